# Initial kernel scaffold; baseline (speedup 1.0000x reference)
import functools

import jax
import jax.numpy as jnp
from jax import lax
from jax.experimental import pallas as pl
from jax.experimental.pallas import tpu as pltpu
from jax.experimental.pallas import tpu_sc as plsc

B = 16
IN_P1 = 10001
OUT_P = 10000
M = 16
W = 9
CIN = 16
COUT = 16
WC = W * CIN
BC = B * CIN

NC, NS = 2, 16
NW = NC * NS
P_PER_W = 320
P_PAD = NW * P_PER_W
CP = 8
NCHUNK = P_PER_W // CP
BG = 4


def _sc_bag_gather(xt, nid_flat, ww_flat):
    mesh = plsc.VectorSubcoreMesh(core_axis_name="c", subcore_axis_name="s")

    @functools.partial(
        pl.kernel,
        mesh=mesh,
        out_type=jax.ShapeDtypeStruct((P_PAD, B * WC), jnp.float32),
        scratch_types=[
            pltpu.VMEM((CP * M,), jnp.int32),
            pltpu.VMEM((CP * M, BC), jnp.float32),
            pltpu.VMEM((CP * M * W,), jnp.float32),
            pltpu.VMEM((CP, B * WC), jnp.float32),
            pltpu.SemaphoreType.DMA,
        ],
    )
    def sc_kernel(xt_hbm, nid_hbm, ww_hbm, g_hbm, idx_v, rows_v, ww_v, g_v, sem):
        wid = lax.axis_index("s") * NC + lax.axis_index("c")
        base_p = wid * P_PER_W

        def compute_body(it, _):
            p = it // BG
            bg = it % BG
            acc = [[jnp.zeros((CIN,), jnp.float32) for _ in range(W)]
                   for _ in range(BG)]
            for m in range(M):
                swid = [ww_v[p * (M * W) + m * W + w] for w in range(W)]
                svec = [lax.broadcast(swid[w], (CIN,)) for w in range(W)]
                for j in range(BG):
                    b = bg * BG + j
                    v = rows_v[p * M + m, pl.ds(b * CIN, CIN)]
                    for w in range(W):
                        acc[j][w] = acc[j][w] + svec[w] * v
            for j in range(BG):
                b = bg * BG + j
                for w in range(W):
                    g_v[p, pl.ds(b * WC + w * CIN, CIN)] = acc[j][w]
            return 0

        def chunk_body(c, _):
            row0 = base_p + c * CP
            pltpu.sync_copy(nid_hbm.at[pl.ds(row0 * M, CP * M)], idx_v)
            pltpu.async_copy(xt_hbm.at[idx_v], rows_v, sem).wait()
            pltpu.sync_copy(ww_hbm.at[pl.ds(row0 * M * W, CP * M * W)], ww_v)
            lax.fori_loop(0, CP * BG, compute_body, 0)
            pltpu.sync_copy(g_v, g_hbm.at[pl.ds(row0, CP)])
            return 0

        lax.fori_loop(0, NCHUNK, chunk_body, 0)

    return sc_kernel(xt, nid_flat, ww_flat)


def _tc_combine(g3, wr, bias2):
    PB = 256
    grid = (P_PAD // PB,)

    def body(g_ref, w_ref, b_ref, o_ref):
        x = g_ref[...].reshape(PB * B, WC)
        y = jnp.dot(x, w_ref[...], preferred_element_type=jnp.float32)
        y = y + b_ref[...]
        o_ref[...] = y.reshape(PB, B, COUT)

    return pl.pallas_call(
        body,
        grid=grid,
        in_specs=[
            pl.BlockSpec((PB, B, WC), lambda i: (i, 0, 0)),
            pl.BlockSpec((WC, COUT), lambda i: (0, 0)),
            pl.BlockSpec((1, COUT), lambda i: (0, 0)),
        ],
        out_specs=pl.BlockSpec((PB, B, COUT), lambda i: (i, 0, 0)),
        out_shape=jax.ShapeDtypeStruct((P_PAD, B, COUT), jnp.float32),
    )(g3, wr, bias2)


def kernel(in_pc_pad, neighbor_id_lstlst, neighbor_mask_lst, weights, bias, w_weights):
    x0 = in_pc_pad.at[:, IN_P1 - 1, :].set(0.0)
    xt = jnp.transpose(x0, (1, 0, 2)).reshape(IN_P1, BC)

    nid_flat = jnp.pad(neighbor_id_lstlst, ((0, P_PAD - OUT_P), (0, 0)),
                       constant_values=IN_P1 - 1).reshape(-1)
    ww_flat = jnp.pad(w_weights, ((0, P_PAD - OUT_P), (0, 0), (0, 0))).reshape(-1)

    g = _sc_bag_gather(xt, nid_flat, ww_flat)
    g3 = g.reshape(P_PAD, B, WC)

    wr = weights.reshape(W, COUT, CIN).transpose(0, 2, 1).reshape(WC, COUT)
    out_t = _tc_combine(g3, wr, bias.reshape(1, COUT))
    return jnp.transpose(out_t, (1, 0, 2))[:, :OUT_P, :]


# speedup vs baseline: 2.4176x; 2.4176x over previous
import functools

import jax
import jax.numpy as jnp
from jax import lax
from jax.experimental import pallas as pl
from jax.experimental.pallas import tpu as pltpu
from jax.experimental.pallas import tpu_sc as plsc

B = 16
IN_P1 = 10001
OUT_P = 10000
M = 16
W = 9
CIN = 16
COUT = 16
WC = W * CIN
BC = B * CIN

NC, NS = 2, 16
NW = NC * NS
P_PER_W = 320
P_PAD = NW * P_PER_W
CP = 8
NCHUNK = P_PER_W // CP
BG = 4


def _sc_bag_gather(xt, nid_flat, ww2):
    mesh = plsc.VectorSubcoreMesh(core_axis_name="c", subcore_axis_name="s")

    @functools.partial(
        pl.kernel,
        mesh=mesh,
        out_type=jax.ShapeDtypeStruct((P_PAD, B * WC), jnp.float32),
        scratch_types=[
            pltpu.VMEM((CP * M,), jnp.int32),
            pltpu.VMEM((CP * M, BC), jnp.float32),
            pltpu.VMEM((CP * M, 16), jnp.float32),
            pltpu.VMEM((CP, B * WC), jnp.float32),
            pltpu.SemaphoreType.DMA,
        ],
    )
    def sc_kernel(xt_hbm, nid_hbm, ww_hbm, g_hbm, idx_v, rows_v, ww_v, g_v, sem):
        wid = lax.axis_index("s") * NC + lax.axis_index("c")
        base_p = wid * P_PER_W

        def compute_body(it, _):
            p = it // BG
            bg = it % BG
            acc = [[jnp.zeros((CIN,), jnp.float32) for _ in range(W)]
                   for _ in range(BG)]
            for m in range(M):
                wvec = ww_v[p * M + m, :]
                svec = [lax.broadcast(wvec[w], (CIN,)) for w in range(W)]
                for j in range(BG):
                    b = bg * BG + j
                    v = rows_v[p * M + m, pl.ds(b * CIN, CIN)]
                    for w in range(W):
                        acc[j][w] = acc[j][w] + svec[w] * v
            for j in range(BG):
                b = bg * BG + j
                for w in range(W):
                    g_v[p, pl.ds(b * WC + w * CIN, CIN)] = acc[j][w]
            return 0

        def chunk_body(c, _):
            row0 = base_p + c * CP
            pltpu.sync_copy(nid_hbm.at[pl.ds(row0 * M, CP * M)], idx_v)
            pltpu.async_copy(xt_hbm.at[idx_v], rows_v, sem).wait()
            pltpu.sync_copy(ww_hbm.at[pl.ds(row0 * M, CP * M)], ww_v)
            lax.fori_loop(0, CP * BG, compute_body, 0)
            pltpu.sync_copy(g_v, g_hbm.at[pl.ds(row0, CP)])
            return 0

        lax.fori_loop(0, NCHUNK, chunk_body, 0)

    return sc_kernel(xt, nid_flat, ww2)


def _tc_combine(g3, wr, bias2):
    PB = 256
    grid = (P_PAD // PB,)

    def body(g_ref, w_ref, b_ref, o_ref):
        x = g_ref[...].reshape(PB * B, WC)
        y = jnp.dot(x, w_ref[...], preferred_element_type=jnp.float32)
        y = y + b_ref[...]
        o_ref[...] = y.reshape(PB, B, COUT)

    return pl.pallas_call(
        body,
        grid=grid,
        in_specs=[
            pl.BlockSpec((PB, B, WC), lambda i: (i, 0, 0)),
            pl.BlockSpec((WC, COUT), lambda i: (0, 0)),
            pl.BlockSpec((1, COUT), lambda i: (0, 0)),
        ],
        out_specs=pl.BlockSpec((PB, B, COUT), lambda i: (i, 0, 0)),
        out_shape=jax.ShapeDtypeStruct((P_PAD, B, COUT), jnp.float32),
    )(g3, wr, bias2)


def kernel(in_pc_pad, neighbor_id_lstlst, neighbor_mask_lst, weights, bias, w_weights):
    x0 = in_pc_pad.at[:, IN_P1 - 1, :].set(0.0)
    xt = jnp.transpose(x0, (1, 0, 2)).reshape(IN_P1, BC)

    nid_flat = jnp.pad(neighbor_id_lstlst, ((0, P_PAD - OUT_P), (0, 0)),
                       constant_values=IN_P1 - 1).reshape(-1)
    ww2 = jnp.pad(w_weights, ((0, P_PAD - OUT_P), (0, 0), (0, 16 - W))
                  ).reshape(P_PAD * M, 16)

    g = _sc_bag_gather(xt, nid_flat, ww2)
    g3 = g.reshape(P_PAD, B, WC)

    wr = weights.reshape(W, COUT, CIN).transpose(0, 2, 1).reshape(WC, COUT)
    out_t = _tc_combine(g3, wr, bias.reshape(1, COUT))
    return jnp.transpose(out_t, (1, 0, 2))[:, :OUT_P, :]


# speedup vs baseline: 4.7058x; 1.9465x over previous
import functools

import jax
import jax.numpy as jnp
from jax import lax
from jax.experimental import pallas as pl
from jax.experimental.pallas import tpu as pltpu
from jax.experimental.pallas import tpu_sc as plsc

B = 16
IN_P1 = 10001
OUT_P = 10000
M = 16
W = 9
CIN = 16
COUT = 16
WC = W * CIN
BC = B * CIN

NC, NS = 2, 16
NW = NC * NS
P_PER_W = 320
P_PAD = NW * P_PER_W
CP = 8
NCHUNK = P_PER_W // CP
BG = 4


def _sc_bag_gather(xt, nid_flat, ww2):
    mesh = plsc.VectorSubcoreMesh(core_axis_name="c", subcore_axis_name="s")

    @functools.partial(
        pl.kernel,
        mesh=mesh,
        out_type=jax.ShapeDtypeStruct((B, P_PAD, WC), jnp.float32),
        scratch_types=[
            pltpu.VMEM((2, CP * M), jnp.int32),
            pltpu.VMEM((2, CP * M, BC), jnp.float32),
            pltpu.VMEM((2, CP, M * W), jnp.float32),
            pltpu.VMEM((B, CP, WC), jnp.float32),
            pltpu.SemaphoreType.DMA,
            pltpu.SemaphoreType.DMA,
        ],
    )
    def sc_kernel(xt_hbm, nid_hbm, ww_hbm, g_hbm, idx_v, rows_v, ww_v, g_v,
                  sem0, sem1):
        wid = lax.axis_index("s") * NC + lax.axis_index("c")
        base_p = wid * P_PER_W
        sems = (sem0, sem1)
        lane_ids = [jnp.full((CIN,), k % 16, jnp.int32) for k in range(M * W)]

        def prefetch(buf, c):
            row0 = base_p + c * CP
            pltpu.sync_copy(nid_hbm.at[pl.ds(row0 * M, CP * M)], idx_v.at[buf])
            cps = [
                pltpu.async_copy(xt_hbm.at[idx_v.at[buf]], rows_v.at[buf],
                                 sems[buf]),
                pltpu.async_copy(ww_hbm.at[pl.ds(row0, CP)],
                                 ww_v.at[buf], sems[buf]),
            ]
            return cps

        def wait(cps):
            for cp in cps:
                cp.wait()

        def make_compute_body(buf):
            def compute_body(it, _):
                p = it // BG
                bg = it % BG
                acc = [[jnp.zeros((CIN,), jnp.float32) for _ in range(W)]
                       for _ in range(BG)]
                wregs = [ww_v[buf, p, pl.ds(j * 16, 16)]
                         for j in range(M * W // 16)]
                for m in range(M):
                    svec = [wregs[(m * W + w) // 16]
                            .at[lane_ids[m * W + w]]
                            .get(mode="promise_in_bounds")
                            for w in range(W)]
                    for j in range(BG):
                        b = bg * BG + j
                        v = rows_v[buf, p * M + m, pl.ds(b * CIN, CIN)]
                        for w in range(W):
                            acc[j][w] = acc[j][w] + svec[w] * v
                for j in range(BG):
                    b = bg * BG + j
                    for w in range(W):
                        g_v[b, p, pl.ds(w * CIN, CIN)] = acc[j][w]
                return 0
            return compute_body

        bodies = (make_compute_body(0), make_compute_body(1))

        def compute(buf, c):
            row0 = base_p + c * CP
            lax.fori_loop(0, CP * BG, bodies[buf], 0)
            pltpu.sync_copy(g_v, g_hbm.at[:, pl.ds(row0, CP), :])

        def pair_body(t, _):
            c0 = 2 * t
            cps1 = prefetch(1, c0 + 1)
            compute(0, c0)
            wait(cps1)
            cps0 = prefetch(0, jnp.minimum(c0 + 2, NCHUNK - 1))
            compute(1, c0 + 1)
            wait(cps0)
            return 0

        wait(prefetch(0, 0))
        lax.fori_loop(0, NCHUNK // 2, pair_body, 0)

    return sc_kernel(xt, nid_flat, ww2)


def _tc_combine(g3, wr, bias2):
    PB = 400
    grid = (OUT_P // PB,)

    def body(g_ref, w_ref, b_ref, o_ref):
        x = g_ref[...].reshape(B * PB, WC)
        y = jnp.dot(x, w_ref[...], preferred_element_type=jnp.float32)
        y = y + b_ref[...]
        o_ref[...] = y.reshape(B, PB, COUT)

    return pl.pallas_call(
        body,
        grid=grid,
        in_specs=[
            pl.BlockSpec((B, PB, WC), lambda i: (0, i, 0)),
            pl.BlockSpec((WC, COUT), lambda i: (0, 0)),
            pl.BlockSpec((1, COUT), lambda i: (0, 0)),
        ],
        out_specs=pl.BlockSpec((B, PB, COUT), lambda i: (0, i, 0)),
        out_shape=jax.ShapeDtypeStruct((B, OUT_P, COUT), jnp.float32),
    )(g3, wr, bias2)


def kernel(in_pc_pad, neighbor_id_lstlst, neighbor_mask_lst, weights, bias, w_weights):
    x0 = in_pc_pad.at[:, IN_P1 - 1, :].set(0.0)
    xt = jnp.transpose(x0, (1, 0, 2)).reshape(IN_P1, BC)

    nid_flat = jnp.pad(neighbor_id_lstlst, ((0, P_PAD - OUT_P), (0, 0)),
                       constant_values=IN_P1 - 1).reshape(-1)
    ww2 = jnp.pad(w_weights, ((0, P_PAD - OUT_P), (0, 0), (0, 0))
                  ).reshape(P_PAD, M * W)

    g3 = _sc_bag_gather(xt, nid_flat, ww2)

    wr = weights.reshape(W, COUT, CIN).transpose(0, 2, 1).reshape(WC, COUT)
    return _tc_combine(g3, wr, bias.reshape(1, COUT))


# speedup vs baseline: 5.2678x; 1.1194x over previous
import functools

import jax
import jax.numpy as jnp
from jax import lax
from jax.experimental import pallas as pl
from jax.experimental.pallas import tpu as pltpu
from jax.experimental.pallas import tpu_sc as plsc

B = 16
IN_P1 = 10001
OUT_P = 10000
M = 16
W = 9
CIN = 16
COUT = 16
WC = W * CIN
BC = B * CIN

NC, NS = 2, 16
NW = NC * NS
P_PER_W = 320
P_PAD = NW * P_PER_W
CP = 4
NCHUNK = P_PER_W // CP
BG = 4


def _sc_bag_gather(xt, nid_flat, ww2):
    mesh = plsc.VectorSubcoreMesh(core_axis_name="c", subcore_axis_name="s")

    @functools.partial(
        pl.kernel,
        mesh=mesh,
        out_type=jax.ShapeDtypeStruct((B, P_PAD, WC), jnp.float32),
        scratch_types=[
            pltpu.VMEM((2, CP * M), jnp.int32),
            pltpu.VMEM((2, CP * M, BC), jnp.float32),
            pltpu.VMEM((2, CP, M * W), jnp.float32),
            pltpu.VMEM((2, B, CP, WC), jnp.float32),
            pltpu.SemaphoreType.DMA,
            pltpu.SemaphoreType.DMA,
            pltpu.SemaphoreType.DMA,
            pltpu.SemaphoreType.DMA,
            pltpu.SemaphoreType.DMA,
            pltpu.SemaphoreType.DMA,
        ],
    )
    def sc_kernel(xt_hbm, nid_hbm, ww_hbm, g_hbm, idx_v, rows_v, ww_v, g_v,
                  si0, si1, sg0, sg1, ss0, ss1):
        wid = lax.axis_index("s") * NC + lax.axis_index("c")
        base_p = wid * P_PER_W
        si, sg, ss = (si0, si1), (sg0, sg1), (ss0, ss1)
        lane_ids = [jnp.full((CIN,), k % 16, jnp.int32) for k in range(M * W)]

        def fire_idx(buf, c):
            row0 = base_p + c * CP
            pltpu.async_copy(nid_hbm.at[pl.ds(row0 * M, CP * M)],
                             idx_v.at[buf], si[buf])

        def wait_idx(buf):
            pltpu.make_async_copy(nid_hbm.at[pl.ds(0, CP * M)],
                                  idx_v.at[buf], si[buf]).wait()

        def fire_gw(buf, c):
            row0 = base_p + c * CP
            pltpu.async_copy(xt_hbm.at[idx_v.at[buf]], rows_v.at[buf],
                             sg[buf])
            pltpu.async_copy(ww_hbm.at[pl.ds(row0, CP)], ww_v.at[buf],
                             sg[buf])

        def wait_gw(buf):
            pltpu.make_async_copy(xt_hbm.at[idx_v.at[buf]], rows_v.at[buf],
                                  sg[buf]).wait()
            pltpu.make_async_copy(ww_hbm.at[pl.ds(0, CP)], ww_v.at[buf],
                                  sg[buf]).wait()

        def fire_store(buf, c):
            row0 = base_p + c * CP
            pltpu.async_copy(g_v.at[buf], g_hbm.at[:, pl.ds(row0, CP), :],
                             ss[buf])

        def wait_store(buf):
            pltpu.make_async_copy(g_v.at[buf],
                                  g_hbm.at[:, pl.ds(base_p, CP), :],
                                  ss[buf]).wait()

        def make_compute_body(buf):
            def compute_body(it, _):
                p = it // BG
                bg = it % BG
                acc = [[jnp.zeros((CIN,), jnp.float32) for _ in range(W)]
                       for _ in range(BG)]
                wregs = [ww_v[buf, p, pl.ds(j * 16, 16)]
                         for j in range(M * W // 16)]
                for m in range(M):
                    svec = [wregs[(m * W + w) // 16]
                            .at[lane_ids[m * W + w]]
                            .get(mode="promise_in_bounds")
                            for w in range(W)]
                    for j in range(BG):
                        b = bg * BG + j
                        v = rows_v[buf, p * M + m, pl.ds(b * CIN, CIN)]
                        for w in range(W):
                            acc[j][w] = acc[j][w] + svec[w] * v
                for j in range(BG):
                    b = bg * BG + j
                    for w in range(W):
                        g_v[buf, b, p, pl.ds(w * CIN, CIN)] = acc[j][w]
                return 0
            return compute_body

        bodies = (make_compute_body(0), make_compute_body(1))

        def compute(buf, c):
            lax.fori_loop(0, CP * BG, bodies[buf], 0)

        def pair_body(t, _):
            c0 = 2 * t
            cn0 = jnp.minimum(c0 + 2, NCHUNK - 1)
            cn1 = jnp.minimum(c0 + 3, NCHUNK - 1)
            wait_gw(0)
            fire_idx(0, cn0)
            wait_store(0)
            compute(0, c0)
            fire_store(0, c0)
            wait_gw(1)
            wait_idx(0)
            fire_gw(0, cn0)
            fire_idx(1, cn1)
            wait_store(1)
            compute(1, c0 + 1)
            fire_store(1, c0 + 1)
            wait_idx(1)
            fire_gw(1, cn1)
            return 0

        fire_idx(0, 0)
        wait_idx(0)
        fire_gw(0, 0)
        fire_idx(1, 1)
        wait_idx(1)
        fire_gw(1, 1)
        fire_store(0, 0)
        fire_store(1, 1)
        lax.fori_loop(0, NCHUNK // 2, pair_body, 0)
        wait_gw(0)
        wait_gw(1)
        wait_store(0)
        wait_store(1)

    return sc_kernel(xt, nid_flat, ww2)


def _tc_combine(g3, wr, bias2):
    PB = 400
    grid = (OUT_P // PB,)

    def body(g_ref, w_ref, b_ref, o_ref):
        x = g_ref[...].reshape(B * PB, WC)
        y = jnp.dot(x, w_ref[...], preferred_element_type=jnp.float32)
        y = y + b_ref[...]
        o_ref[...] = y.reshape(B, PB, COUT)

    return pl.pallas_call(
        body,
        grid=grid,
        in_specs=[
            pl.BlockSpec((B, PB, WC), lambda i: (0, i, 0)),
            pl.BlockSpec((WC, COUT), lambda i: (0, 0)),
            pl.BlockSpec((1, COUT), lambda i: (0, 0)),
        ],
        out_specs=pl.BlockSpec((B, PB, COUT), lambda i: (0, i, 0)),
        out_shape=jax.ShapeDtypeStruct((B, OUT_P, COUT), jnp.float32),
    )(g3, wr, bias2)


def kernel(in_pc_pad, neighbor_id_lstlst, neighbor_mask_lst, weights, bias, w_weights):
    x0 = in_pc_pad.at[:, IN_P1 - 1, :].set(0.0)
    xt = jnp.transpose(x0, (1, 0, 2)).reshape(IN_P1, BC)

    nid_flat = jnp.pad(neighbor_id_lstlst, ((0, P_PAD - OUT_P), (0, 0)),
                       constant_values=IN_P1 - 1).reshape(-1)
    ww2 = jnp.pad(w_weights, ((0, P_PAD - OUT_P), (0, 0), (0, 0))
                  ).reshape(P_PAD, M * W)

    g3 = _sc_bag_gather(xt, nid_flat, ww2)

    wr = weights.reshape(W, COUT, CIN).transpose(0, 2, 1).reshape(WC, COUT)
    return _tc_combine(g3, wr, bias.reshape(1, COUT))
